# Initial kernel scaffold; baseline (speedup 1.0000x reference)
import jax
import jax.numpy as jnp
from jax.experimental import pallas as pl


def kernel(hidden_states, weight):
    raise NotImplementedError("write your pallas kernel here")


# speedup vs baseline: 1.0644x; 1.0644x over previous
import functools

import jax
import jax.numpy as jnp
from jax.experimental import pallas as pl
from jax.experimental.pallas import tpu as pltpu

_TOP_K = 8
_NUM_EXPERTS = 64
_HIDDEN = 4096
_BLOCK_M = 512


def _router_body(x_ref, wt_ref, probs_ref, scores_ref, idx_ref):
    x = x_ref[...]
    wt = wt_ref[...]
    logits = jax.lax.dot_general(
        x, wt, (((1,), (0,)), ((), ())), preferred_element_type=jnp.float32
    )
    m = jnp.max(logits, axis=-1, keepdims=True)
    e = jnp.exp(logits - m)
    probs = e / jnp.sum(e, axis=-1, keepdims=True)
    probs_ref[...] = probs

    iota = jax.lax.broadcasted_iota(jnp.int32, probs.shape, 1)
    work = probs
    vals = []
    idxs = []
    for _ in range(_TOP_K):
        mx = jnp.max(work, axis=-1, keepdims=True)
        is_mx = work == mx
        ix = jnp.min(jnp.where(is_mx, iota, _NUM_EXPERTS), axis=-1, keepdims=True)
        vals.append(mx)
        idxs.append(ix)
        work = jnp.where(iota == ix, -1.0, work)
    top_vals = jnp.concatenate(vals, axis=-1)
    top_idx = jnp.concatenate(idxs, axis=-1)
    scores_ref[...] = top_vals / jnp.sum(top_vals, axis=-1, keepdims=True)
    idx_ref[...] = top_idx


@jax.jit
def kernel(hidden_states, weight):
    x = hidden_states.reshape(-1, _HIDDEN)
    n_tokens = x.shape[0]
    wt = weight.T
    grid = (n_tokens // _BLOCK_M,)
    probs, scores, idx = pl.pallas_call(
        _router_body,
        grid=grid,
        in_specs=[
            pl.BlockSpec((_BLOCK_M, _HIDDEN), lambda i: (i, 0)),
            pl.BlockSpec((_HIDDEN, _NUM_EXPERTS), lambda i: (0, 0)),
        ],
        out_specs=[
            pl.BlockSpec((_BLOCK_M, _NUM_EXPERTS), lambda i: (i, 0)),
            pl.BlockSpec((_BLOCK_M, _TOP_K), lambda i: (i, 0)),
            pl.BlockSpec((_BLOCK_M, _TOP_K), lambda i: (i, 0)),
        ],
        out_shape=[
            jax.ShapeDtypeStruct((n_tokens, _NUM_EXPERTS), jnp.float32),
            jax.ShapeDtypeStruct((n_tokens, _TOP_K), jnp.float32),
            jax.ShapeDtypeStruct((n_tokens, _TOP_K), jnp.int32),
        ],
    )(x, wt)
    return (probs, scores, idx)


# speedup vs baseline: 1.1777x; 1.1065x over previous
import functools

import jax
import jax.numpy as jnp
from jax.experimental import pallas as pl
from jax.experimental.pallas import tpu as pltpu

_TOP_K = 8
_NUM_EXPERTS = 64
_HIDDEN = 4096
_BLOCK_M = 1024


def _router_body(x_ref, wt_ref, probs_ref, scores_ref, idx_ref):
    x = x_ref[...]
    wt = wt_ref[...]
    logits = jax.lax.dot_general(
        x, wt, (((1,), (0,)), ((), ())), preferred_element_type=jnp.float32
    )
    m = jnp.max(logits, axis=-1, keepdims=True)
    e = jnp.exp(logits - m)
    probs = e / jnp.sum(e, axis=-1, keepdims=True)
    probs_ref[...] = probs

    iota = jax.lax.broadcasted_iota(jnp.int32, probs.shape, 1)
    work = probs
    vals = []
    idxs = []
    for _ in range(_TOP_K):
        mx = jnp.max(work, axis=-1, keepdims=True)
        is_mx = work == mx
        ix = jnp.min(jnp.where(is_mx, iota, _NUM_EXPERTS), axis=-1, keepdims=True)
        vals.append(mx)
        idxs.append(ix)
        work = jnp.where(iota == ix, -1.0, work)
    top_vals = jnp.concatenate(vals, axis=-1)
    top_idx = jnp.concatenate(idxs, axis=-1)
    scores_ref[...] = top_vals / jnp.sum(top_vals, axis=-1, keepdims=True)
    idx_ref[...] = top_idx


@jax.jit
def kernel(hidden_states, weight):
    x = hidden_states.reshape(-1, _HIDDEN)
    n_tokens = x.shape[0]
    wt = weight.T
    grid = (n_tokens // _BLOCK_M,)
    probs, scores, idx = pl.pallas_call(
        _router_body,
        grid=grid,
        in_specs=[
            pl.BlockSpec((_BLOCK_M, _HIDDEN), lambda i: (i, 0)),
            pl.BlockSpec((_HIDDEN, _NUM_EXPERTS), lambda i: (0, 0)),
        ],
        out_specs=[
            pl.BlockSpec((_BLOCK_M, _NUM_EXPERTS), lambda i: (i, 0)),
            pl.BlockSpec((_BLOCK_M, _TOP_K), lambda i: (i, 0)),
            pl.BlockSpec((_BLOCK_M, _TOP_K), lambda i: (i, 0)),
        ],
        out_shape=[
            jax.ShapeDtypeStruct((n_tokens, _NUM_EXPERTS), jnp.float32),
            jax.ShapeDtypeStruct((n_tokens, _TOP_K), jnp.float32),
            jax.ShapeDtypeStruct((n_tokens, _TOP_K), jnp.int32),
        ],
    )(x, wt)
    return (probs, scores, idx)
